# baseline (speedup 1.0000x reference)
import functools

import jax
import jax.numpy as jnp
from jax import lax
from jax.experimental import pallas as pl
from jax.experimental.pallas import tpu as pltpu
from jax.experimental.pallas import tpu_sc as plsc

SEQ_LEN = 2048
HIDDEN = 1024
BASE = 10000.0
BATCH = 4

NC, NS, L = 2, 16, 16
NW = NC * NS
TOKENS = BATCH * SEQ_LEN
TOK_PER_W = TOKENS // NW
CHUNK = 32
N_CHUNKS = TOK_PER_W // CHUNK


def _rope_tables_signed():
    i = jnp.arange(0, HIDDEN, 2, dtype=jnp.float32)
    theta = (BASE ** (-2.0 * i / HIDDEN))[None, :]
    pos = jnp.arange(0, SEQ_LEN, dtype=jnp.float32)[:, None]
    m_theta = pos @ theta
    cos = jnp.repeat(jnp.cos(m_theta), 2, axis=1)
    sin = jnp.repeat(jnp.sin(m_theta), 2, axis=1)
    sign = jnp.tile(jnp.array([-1.0, 1.0], jnp.float32), HIDDEN // 2)
    return cos, sin * sign[None, :]


def _rope_body(x_hbm, table_hbm, cos_hbm, sin_hbm, out_hbm,
               idx_v, rows_v, cos_v, sin_v, sem):
    wid = lax.axis_index("s") * NC + lax.axis_index("c")
    base = wid * TOK_PER_W
    pos_base = (wid % (SEQ_LEN // TOK_PER_W)) * TOK_PER_W

    lane = lax.iota(jnp.int32, L)
    perm = lane ^ 1

    def chunk_step(k, _):
        tok0 = base + k * CHUNK
        pos0 = pos_base + k * CHUNK
        pltpu.sync_copy(x_hbm.at[pl.ds(tok0, CHUNK)], idx_v)
        pltpu.async_copy(table_hbm.at[idx_v], rows_v, sem).wait()
        pltpu.sync_copy(cos_hbm.at[pl.ds(pos0, CHUNK)], cos_v)
        pltpu.sync_copy(sin_hbm.at[pl.ds(pos0, CHUNK)], sin_v)

        def row_step(r, _):
            rr = jnp.full((L,), r, jnp.int32)

            def col_step(c, _):
                for u in range(4):
                    c0 = (c * 4 + u) * L
                    e = rows_v[r, pl.ds(c0, L)]
                    p = plsc.load_gather(rows_v, [rr, c0 + perm])
                    rows_v[r, pl.ds(c0, L)] = (
                        e * cos_v[r, pl.ds(c0, L)] + p * sin_v[r, pl.ds(c0, L)]
                    )
                return 0

            lax.fori_loop(0, HIDDEN // (4 * L), col_step, 0, unroll=False)
            return 0

        lax.fori_loop(0, CHUNK, row_step, 0, unroll=False)
        pltpu.sync_copy(rows_v, out_hbm.at[pl.ds(tok0, CHUNK)])
        return 0

    lax.fori_loop(0, N_CHUNKS, chunk_step, 0, unroll=False)


@functools.partial(jax.jit, static_argnames=())
def kernel(x, table):
    cos, sin_s = _rope_tables_signed()
    x_flat = x.reshape(TOKENS)
    mesh = plsc.VectorSubcoreMesh(
        core_axis_name="c", subcore_axis_name="s", num_cores=NC, num_subcores=NS
    )
    run = pl.kernel(
        _rope_body,
        out_type=jax.ShapeDtypeStruct((TOKENS, HIDDEN), jnp.float32),
        mesh=mesh,
        scratch_types=[
            pltpu.VMEM((CHUNK,), jnp.int32),
            pltpu.VMEM((CHUNK, HIDDEN), jnp.float32),
            pltpu.VMEM((CHUNK, HIDDEN), jnp.float32),
            pltpu.VMEM((CHUNK, HIDDEN), jnp.float32),
            pltpu.SemaphoreType.DMA,
        ],
        compiler_params=pltpu.CompilerParams(
            use_tc_tiling_on_sc=False, needs_layout_passes=False
        ),
    )
    out = run(x_flat, table, cos, sin_s)
    return out.reshape(BATCH, SEQ_LEN, HIDDEN)


# speedup vs baseline: 3.5283x; 3.5283x over previous
import functools

import jax
import jax.numpy as jnp
from jax import lax
from jax.experimental import pallas as pl
from jax.experimental.pallas import tpu as pltpu
from jax.experimental.pallas import tpu_sc as plsc

SEQ_LEN = 2048
HIDDEN = 1024
BASE = 10000.0
BATCH = 4

NC, NS, L = 2, 16, 16
NW = NC * NS
POS_PER_W = SEQ_LEN // NW
PCHUNK = 16
N_PCHUNKS = POS_PER_W // PCHUNK


def _rope_tables_signed():
    i = jnp.arange(0, HIDDEN, 2, dtype=jnp.float32)
    theta = (BASE ** (-2.0 * i / HIDDEN))[None, :]
    pos = jnp.arange(0, SEQ_LEN, dtype=jnp.float32)[:, None]
    m_theta = pos @ theta
    cos = jnp.repeat(jnp.cos(m_theta), 2, axis=1)
    sin = jnp.repeat(jnp.sin(m_theta), 2, axis=1)
    sign = jnp.tile(jnp.array([-1.0, 1.0], jnp.float32), HIDDEN // 2)
    return cos, sin * sign[None, :]


def _pair_swap(e, perm):
    dn = lax.GatherDimensionNumbers(
        offset_dims=(), collapsed_slice_dims=(0,), start_index_map=(0,)
    )
    return lax.gather(e, perm[:, None], dn, slice_sizes=(1,),
                      mode=lax.GatherScatterMode.PROMISE_IN_BOUNDS)


def _rope_body(x_hbm, table_hbm, cos_hbm, sin_hbm, out_hbm,
               idx_v, rows_v, cos_v, sin_v, sem):
    wid = lax.axis_index("s") * NC + lax.axis_index("c")
    pos_w = wid * POS_PER_W

    perm = lax.iota(jnp.int32, L) ^ 1

    def pchunk_step(p, _):
        pos0 = pos_w + p * PCHUNK
        pltpu.sync_copy(cos_hbm.at[pl.ds(pos0, PCHUNK)], cos_v)
        pltpu.sync_copy(sin_hbm.at[pl.ds(pos0, PCHUNK)], sin_v)

        def batch_step(b, _):
            pltpu.sync_copy(x_hbm.at[b, pl.ds(pos0, PCHUNK)], idx_v)
            pltpu.async_copy(table_hbm.at[idx_v], rows_v, sem).wait()

            def row_step(r, _):
                def col_step(c, _):
                    for u in range(4):
                        c0 = (c * 4 + u) * L
                        e = rows_v[r, pl.ds(c0, L)]
                        pv = _pair_swap(e, perm)
                        rows_v[r, pl.ds(c0, L)] = (
                            e * cos_v[r, pl.ds(c0, L)]
                            + pv * sin_v[r, pl.ds(c0, L)]
                        )
                    return 0

                lax.fori_loop(0, HIDDEN // (4 * L), col_step, 0, unroll=False)
                return 0

            lax.fori_loop(0, PCHUNK, row_step, 0, unroll=False)
            pltpu.sync_copy(rows_v, out_hbm.at[b, pl.ds(pos0, PCHUNK)])
            return 0

        lax.fori_loop(0, BATCH, batch_step, 0, unroll=False)
        return 0

    lax.fori_loop(0, N_PCHUNKS, pchunk_step, 0, unroll=False)


@functools.partial(jax.jit, static_argnames=())
def kernel(x, table):
    cos, sin_s = _rope_tables_signed()
    mesh = plsc.VectorSubcoreMesh(
        core_axis_name="c", subcore_axis_name="s", num_cores=NC, num_subcores=NS
    )
    run = pl.kernel(
        _rope_body,
        out_type=jax.ShapeDtypeStruct((BATCH, SEQ_LEN, HIDDEN), jnp.float32),
        mesh=mesh,
        scratch_types=[
            pltpu.VMEM((PCHUNK,), jnp.int32),
            pltpu.VMEM((PCHUNK, HIDDEN), jnp.float32),
            pltpu.VMEM((PCHUNK, HIDDEN), jnp.float32),
            pltpu.VMEM((PCHUNK, HIDDEN), jnp.float32),
            pltpu.SemaphoreType.DMA,
        ],
        compiler_params=pltpu.CompilerParams(needs_layout_passes=False),
    )
    return run(x, table, cos, sin_s)
